# Initial kernel scaffold; baseline (speedup 1.0000x reference)
import jax
import jax.numpy as jnp
from jax.experimental import pallas as pl


def kernel(x, W1, b1, g1, be1, W2, b2, g2, be2, W3, b3):
    raise NotImplementedError("write your pallas kernel here")


# speedup vs baseline: 1.1596x; 1.1596x over previous
import functools

import jax
import jax.numpy as jnp
from jax.experimental import pallas as pl
from jax.experimental.pallas import tpu as pltpu

_BM = 512
_H = 1024
_BN_EPS = 1e-5

_DP = (-10.0, -2.5, -1.0, 1.0, 2.5, 10.0)
_KP = (0.5, 0.3417968, 0.3417968, 0.3417968, 0.3417968, 0.5)
_KI0 = 0.1503906
_KD = 0.0097656


def _mlp_kernel(x_ref, w1_ref, w2_ref, w3_ref, o_ref):
    xb = x_ref[...]
    lo_p = xb[8:9, :]
    hi_p = xb[9:10, :]
    aim_lo = xb[10:11, :]
    aim_hi = xb[11:12, :]
    kp_rate = xb[15:16, :]

    diff_hi = hi_p - aim_hi
    d = diff_hi / 1000.0

    seg0 = _KP[0] + (d - _DP[0]) * jnp.float32(_KP[1] - _KP[0]) / (_DP[1] - _DP[0])
    seg4 = _KP[4] + (d - _DP[4]) * jnp.float32(_KP[5] - _KP[4]) / (_DP[5] - _DP[4])
    kp_val = jnp.where(d <= _DP[1], seg0,
                       jnp.where(d > _DP[4], seg4, jnp.float32(_KP[1])))

    kp = kp_val * kp_rate
    ki = jnp.float32(_KI0) * kp_rate
    kd = jnp.float32(_KD) * kp_rate

    ones = jnp.ones((8, xb.shape[1]), jnp.float32)
    ut = jnp.concatenate(
        [kp, ki, kd, diff_hi, lo_p, hi_p, aim_lo, aim_hi, ones], axis=0)

    z1 = jnp.dot(w1_ref[...], ut, preferred_element_type=jnp.float32)
    h1 = jnp.tanh(z1)
    h1a = jnp.concatenate([h1, ones], axis=0)
    z2 = jnp.dot(w2_ref[...], h1a, preferred_element_type=jnp.float32)
    h2 = jnp.tanh(z2)
    h2a = jnp.concatenate([h2, ones], axis=0)
    o8 = jnp.dot(w3_ref[...], h2a, preferred_element_type=jnp.float32)
    o_ref[...] = o8[0:1, :]


@jax.jit
def kernel(x, W1, b1, g1, be1, W2, b2, g2, be2, W3, b3):
    B = x.shape[0]
    rs = jax.lax.rsqrt(jnp.float32(1.0 + _BN_EPS))

    s1 = g1 * rs
    c1 = b1 * s1 + be1
    s2 = g2 * rs
    c2 = b2 * s2 + be2
    w1aug = jnp.concatenate(
        [W1 * s1[:, None], c1[:, None], jnp.zeros((_H, 7), jnp.float32)], axis=1)
    w2aug = jnp.concatenate(
        [W2 * s2[:, None], c2[:, None], jnp.zeros((_H, 7), jnp.float32)], axis=1)
    w3row = jnp.concatenate([W3[0], b3, jnp.zeros((7,), jnp.float32)])
    w3aug = jnp.zeros((8, 1032), jnp.float32).at[0].set(w3row)

    xt = x.T
    grid = (B // _BM,)

    out = pl.pallas_call(
        _mlp_kernel,
        grid=grid,
        in_specs=[
            pl.BlockSpec((16, _BM), lambda i: (0, i)),
            pl.BlockSpec((_H, 16), lambda i: (0, 0)),
            pl.BlockSpec((_H, 1032), lambda i: (0, 0)),
            pl.BlockSpec((8, 1032), lambda i: (0, 0)),
        ],
        out_specs=pl.BlockSpec((1, _BM), lambda i: (0, i)),
        out_shape=jax.ShapeDtypeStruct((1, B), jnp.float32),
        compiler_params=pltpu.CompilerParams(
            dimension_semantics=("parallel",),
        ),
    )(xt, w1aug, w2aug, w3aug)
    return out[0]
